# Initial kernel scaffold; baseline (speedup 1.0000x reference)
import jax
import jax.numpy as jnp
from jax.experimental import pallas as pl


def kernel(src, edge_index, W_g1, b_g1, W_g2, b_g2, conv1_w, conv1_b, conv2_w, conv2_b, conv3_w, conv3_b, fc1_w, fc1_b, cls1_w, cls1_b, cls2_w, cls2_b):
    raise NotImplementedError("write your pallas kernel here")


# speedup vs baseline: 11.1978x; 11.1978x over previous
import jax
import jax.numpy as jnp
from jax.experimental import pallas as pl
from jax.experimental.pallas import tpu as pltpu

B, N, E = 256, 192, 3072
C_IN = 128
GB = 32


def _adj_body(ei_ref, a_ref):
    s = ei_ref[:, 0:1]
    d = ei_ref[:, 1:2]
    lane = jax.lax.broadcasted_iota(jnp.int32, (E, N), 1)
    S1 = (lane == s).astype(jnp.float32)
    D1 = (lane == d).astype(jnp.float32)
    dims = (((0,), (0,)), ((), ()))
    C = jax.lax.dot_general(D1, S1, dims, preferred_element_type=jnp.float32)
    CT = jax.lax.dot_general(S1, D1, dims, preferred_element_type=jnp.float32)
    r = jax.lax.broadcasted_iota(jnp.int32, (N, N), 0)
    c = jax.lax.broadcasted_iota(jnp.int32, (N, N), 1)
    eye = (r == c).astype(jnp.float32)
    C = C + eye
    CT = CT + eye
    deg_col = jnp.sum(C, axis=1, keepdims=True)
    deg_row = jnp.sum(CT, axis=0, keepdims=True)
    a_ref[...] = C * jax.lax.rsqrt(deg_col) * jax.lax.rsqrt(deg_row)


def _build_adj(edge_index):
    ei_t = edge_index.T.astype(jnp.int32)
    return pl.pallas_call(
        _adj_body,
        out_shape=jax.ShapeDtypeStruct((N, N), jnp.float32),
    )(ei_t)


def _conv_s2(X, w_ref, b_ref, K, pad):
    T = X.shape[0] // 2
    gb, Cin = X.shape[1], X.shape[2]
    Cout = w_ref.shape[2]
    Xp = X.reshape(T, 2, gb, Cin)
    z = jnp.zeros((2, 2, gb, Cin), X.dtype)
    Xpad = jnp.concatenate([z, Xp, z], axis=0)
    acc = None
    for k in range(K):
        j = k - pad
        m, r = j // 2, j % 2
        sl = Xpad[2 + m:2 + m + T, r]
        term = jnp.dot(sl.reshape(T * gb, Cin), w_ref[k],
                       preferred_element_type=jnp.float32)
        acc = term if acc is None else acc + term
    acc = acc.reshape(T, gb, Cout) + b_ref[...].reshape(1, 1, Cout)
    return jnp.maximum(acc, 0.0)


def _main_body(src_ref, a_ref, w1_ref, b1_ref, w2_ref, b2_ref,
               cw1_ref, cb1_ref, cw2_ref, cb2_ref, cw3_ref, cb3_ref,
               out_ref, x4_s):
    A = a_ref[...]
    w1 = w1_ref[...]
    w2 = w2_ref[...]
    b1 = b1_ref[...]
    b2 = b2_ref[...]
    for b in range(GB):
        h = jax.lax.dot_general(
            src_ref[b], w1, (((0,), (0,)), ((), ())),
            preferred_element_type=jnp.float32)
        h = jnp.maximum(jnp.dot(A, h, preferred_element_type=jnp.float32) + b1, 0.0)
        h = jnp.dot(h, w2, preferred_element_type=jnp.float32)
        h = jnp.maximum(jnp.dot(A, h, preferred_element_type=jnp.float32) + b2, 0.0)
        x4_s[:, b, :] = h
    x4 = x4_s[...]
    y = _conv_s2(x4, cw1_ref, cb1_ref, K=7, pad=3)
    y = _conv_s2(y, cw2_ref, cb2_ref, K=5, pad=2)
    y = _conv_s2(y, cw3_ref, cb3_ref, K=3, pad=1)
    out_ref[...] = y


def _run_main(src, A, W_g1, b_g1, W_g2, b_g2, cw1, cb1, cw2, cb2, cw3, cb3):
    nsteps = B // GB
    full = lambda shape: pl.BlockSpec(shape, lambda i: (0,) * len(shape))
    return pl.pallas_call(
        _main_body,
        grid=(nsteps,),
        in_specs=[
            pl.BlockSpec((GB, C_IN, N), lambda i: (i, 0, 0)),
            full((N, N)),
            full((C_IN, 64)), full((1, 64)),
            full((64, 32)), full((1, 32)),
            full((7, 32, 32)), full((1, 32)),
            full((5, 32, 64)), full((1, 64)),
            full((3, 64, 128)), full((1, 128)),
        ],
        out_specs=pl.BlockSpec((24, GB, 128), lambda i: (0, i, 0)),
        out_shape=jax.ShapeDtypeStruct((24, B, 128), jnp.float32),
        scratch_shapes=[pltpu.VMEM((N, GB, 32), jnp.float32)],
    )(src, A, W_g1, b_g1.reshape(1, 64), W_g2, b_g2.reshape(1, 32),
      cw1, cb1.reshape(1, 32), cw2, cb2.reshape(1, 64), cw3, cb3.reshape(1, 128))


def _mlp_body(f_ref, wr_ref, fb_ref, c1w_ref, c1b_ref, c2w_ref, c2b_ref,
              logits_ref, feat_ref):
    acc = jnp.zeros((B, 256), jnp.float32)
    for t in range(24):
        acc = acc + jnp.dot(f_ref[t], wr_ref[t],
                            preferred_element_type=jnp.float32)
    feat = acc + fb_ref[...]
    feat_ref[...] = feat
    h = jnp.maximum(feat, 0.0)
    h = jnp.maximum(jnp.dot(h, c1w_ref[...],
                            preferred_element_type=jnp.float32) + c1b_ref[...], 0.0)
    logits_ref[...] = jnp.dot(h, c2w_ref[...],
                              preferred_element_type=jnp.float32) + c2b_ref[...]


def _run_mlp(feats, wr, fc1_b, cls1_w, cls1_b, cls2_w, cls2_b):
    return pl.pallas_call(
        _mlp_body,
        out_shape=(jax.ShapeDtypeStruct((B, 210), jnp.float32),
                   jax.ShapeDtypeStruct((B, 256), jnp.float32)),
    )(feats, wr, fc1_b.reshape(1, 256), cls1_w, cls1_b.reshape(1, 256),
      cls2_w, cls2_b.reshape(1, 210))


def kernel(src, edge_index, W_g1, b_g1, W_g2, b_g2, conv1_w, conv1_b,
           conv2_w, conv2_b, conv3_w, conv3_b, fc1_w, fc1_b,
           cls1_w, cls1_b, cls2_w, cls2_b):
    A = _build_adj(edge_index)
    cw1 = conv1_w.transpose(2, 1, 0)
    cw2 = conv2_w.transpose(2, 1, 0)
    cw3 = conv3_w.transpose(2, 1, 0)
    wr = fc1_w.reshape(128, 24, 256).transpose(1, 0, 2)
    feats = _run_main(src, A, W_g1, b_g1, W_g2, b_g2,
                      cw1, conv1_b, cw2, conv2_b, cw3, conv3_b)
    logits, feat = _run_mlp(feats, wr, fc1_b, cls1_w, cls1_b, cls2_w, cls2_b)
    return (logits, feat)


# speedup vs baseline: 14.5803x; 1.3021x over previous
import jax
import jax.numpy as jnp
from jax.experimental import pallas as pl
from jax.experimental.pallas import tpu as pltpu

B, N, E = 256, 192, 3072
C_IN = 128
GB = 32


def _adj_body(ei_ref, a_ref):
    s = ei_ref[:, 0:1]
    d = ei_ref[:, 1:2]
    lane = jax.lax.broadcasted_iota(jnp.int32, (E, N), 1)
    S1 = (lane == s).astype(jnp.float32)
    D1 = (lane == d).astype(jnp.float32)
    dims = (((0,), (0,)), ((), ()))
    C = jax.lax.dot_general(D1, S1, dims, preferred_element_type=jnp.float32)
    CT = jax.lax.dot_general(S1, D1, dims, preferred_element_type=jnp.float32)
    r = jax.lax.broadcasted_iota(jnp.int32, (N, N), 0)
    c = jax.lax.broadcasted_iota(jnp.int32, (N, N), 1)
    eye = (r == c).astype(jnp.float32)
    C = C + eye
    CT = CT + eye
    deg_col = jnp.sum(C, axis=1, keepdims=True)
    deg_row = jnp.sum(CT, axis=0, keepdims=True)
    a_ref[...] = CT * jax.lax.rsqrt(deg_col) * jax.lax.rsqrt(deg_row)


def _build_adj(edge_index):
    ei_t = edge_index.T.astype(jnp.int32)
    return pl.pallas_call(
        _adj_body,
        out_shape=jax.ShapeDtypeStruct((N, N), jnp.float32),
    )(ei_t)


def _conv_s2(X, w_ref, b_ref, K, pad):
    T = X.shape[0] // 2
    gb, Cin = X.shape[1], X.shape[2]
    Cout = w_ref.shape[2]
    Xp = X.reshape(T, 2, gb, Cin)
    z = jnp.zeros((2, 2, gb, Cin), X.dtype)
    Xpad = jnp.concatenate([z, Xp, z], axis=0)
    acc = None
    for k in range(K):
        j = k - pad
        m, r = j // 2, j % 2
        sl = Xpad[2 + m:2 + m + T, r]
        term = jnp.dot(sl.reshape(T * gb, Cin), w_ref[k],
                       preferred_element_type=jnp.float32)
        acc = term if acc is None else acc + term
    acc = acc.reshape(T, gb, Cout) + b_ref[...].reshape(1, 1, Cout)
    return jnp.maximum(acc, 0.0)


def _main_body(src_ref, a2_ref, w1_ref, b1t_ref, w2_ref, b2t_ref,
               cw1_ref, cb1_ref, cw2_ref, cb2_ref, cw3_ref, cb3_ref,
               out_ref, s1t, s2t, x4_s):
    A2 = a2_ref[...]
    w1 = w1_ref[...]
    w2 = w2_ref[...]
    c0 = (((0,), (0,)), ((), ()))
    for b in range(GB):
        s1t[64 * b:64 * (b + 1), :] = jax.lax.dot_general(
            w1, src_ref[b], c0, preferred_element_type=jnp.float32)
    s1t[...] = jnp.maximum(
        jnp.dot(s1t[...], A2, preferred_element_type=jnp.float32)
        + b1t_ref[...], 0.0)
    for b in range(GB):
        s2t[32 * b:32 * (b + 1), :] = jax.lax.dot_general(
            w2, s1t[64 * b:64 * (b + 1), :], c0,
            preferred_element_type=jnp.float32)
    s2t[...] = jnp.maximum(
        jnp.dot(s2t[...], A2, preferred_element_type=jnp.float32)
        + b2t_ref[...], 0.0)
    for b in range(GB):
        x4_s[:, b, :] = s2t[32 * b:32 * (b + 1), :].T
    x4 = x4_s[...]
    y = _conv_s2(x4, cw1_ref, cb1_ref, K=7, pad=3)
    y = _conv_s2(y, cw2_ref, cb2_ref, K=5, pad=2)
    y = _conv_s2(y, cw3_ref, cb3_ref, K=3, pad=1)
    out_ref[...] = y


def _run_main(src, A, W_g1, b_g1, W_g2, b_g2, cw1, cb1, cw2, cb2, cw3, cb3):
    nsteps = B // GB
    full = lambda shape: pl.BlockSpec(shape, lambda i: (0,) * len(shape))
    return pl.pallas_call(
        _main_body,
        grid=(nsteps,),
        in_specs=[
            pl.BlockSpec((GB, C_IN, N), lambda i: (i, 0, 0)),
            full((N, N)),
            full((C_IN, 64)), full((GB * 64, 1)),
            full((64, 32)), full((GB * 32, 1)),
            full((7, 32, 32)), full((1, 32)),
            full((5, 32, 64)), full((1, 64)),
            full((3, 64, 128)), full((1, 128)),
        ],
        out_specs=pl.BlockSpec((24, GB, 128), lambda i: (0, i, 0)),
        out_shape=jax.ShapeDtypeStruct((24, B, 128), jnp.float32),
        scratch_shapes=[pltpu.VMEM((GB * 64, N), jnp.float32),
                        pltpu.VMEM((GB * 32, N), jnp.float32),
                        pltpu.VMEM((N, GB, 32), jnp.float32)],
    )(src, A, W_g1, jnp.tile(b_g1, GB).reshape(GB * 64, 1),
      W_g2, jnp.tile(b_g2, GB).reshape(GB * 32, 1),
      cw1, cb1.reshape(1, 32), cw2, cb2.reshape(1, 64), cw3, cb3.reshape(1, 128))


def _mlp_body(f_ref, wr_ref, fb_ref, c1w_ref, c1b_ref, c2w_ref, c2b_ref,
              logits_ref, feat_ref):
    acc = jnp.zeros((B, 256), jnp.float32)
    for t in range(24):
        acc = acc + jnp.dot(f_ref[t], wr_ref[t],
                            preferred_element_type=jnp.float32)
    feat = acc + fb_ref[...]
    feat_ref[...] = feat
    h = jnp.maximum(feat, 0.0)
    h = jnp.maximum(jnp.dot(h, c1w_ref[...],
                            preferred_element_type=jnp.float32) + c1b_ref[...], 0.0)
    logits_ref[...] = jnp.dot(h, c2w_ref[...],
                              preferred_element_type=jnp.float32) + c2b_ref[...]


def _run_mlp(feats, wr, fc1_b, cls1_w, cls1_b, cls2_w, cls2_b):
    return pl.pallas_call(
        _mlp_body,
        out_shape=(jax.ShapeDtypeStruct((B, 210), jnp.float32),
                   jax.ShapeDtypeStruct((B, 256), jnp.float32)),
    )(feats, wr, fc1_b.reshape(1, 256), cls1_w, cls1_b.reshape(1, 256),
      cls2_w, cls2_b.reshape(1, 210))


def kernel(src, edge_index, W_g1, b_g1, W_g2, b_g2, conv1_w, conv1_b,
           conv2_w, conv2_b, conv3_w, conv3_b, fc1_w, fc1_b,
           cls1_w, cls1_b, cls2_w, cls2_b):
    A = _build_adj(edge_index)
    cw1 = conv1_w.transpose(2, 1, 0)
    cw2 = conv2_w.transpose(2, 1, 0)
    cw3 = conv3_w.transpose(2, 1, 0)
    wr = fc1_w.reshape(128, 24, 256).transpose(1, 0, 2)
    feats = _run_main(src, A, W_g1, b_g1, W_g2, b_g2,
                      cw1, conv1_b, cw2, conv2_b, cw3, conv3_b)
    logits, feat = _run_mlp(feats, wr, fc1_b, cls1_w, cls1_b, cls2_w, cls2_b)
    return (logits, feat)
